# baseline (speedup 1.0000x reference)
import functools

import jax
import jax.numpy as jnp
from jax import lax
from jax.experimental import pallas as pl
from jax.experimental.pallas import tpu as pltpu
from jax.experimental.pallas import tpu_sc as plsc

N_BASES = 4
L = 1024
BI = 256
NW = 32
EW = L // NW
VL = 16


def _sc_gather(seq_hbm, tab_hbm, oht_hbm, seq_v, tab_v, ohtbuf_v, sem):
    wid = lax.axis_index("s") * 2 + lax.axis_index("c")
    base = wid * EW
    cp_seq = pltpu.async_copy(seq_hbm.at[pl.ds(base, EW)], seq_v, sem)
    cp_tab = pltpu.async_copy(tab_hbm, tab_v, sem)
    cp_seq.wait()
    cp_tab.wait()
    lanes = lax.iota(jnp.int32, VL)
    for c in range(N_BASES):
        for g in range(EW // VL):
            seq16 = seq_v[pl.ds(g * VL, VL)]
            ohtbuf_v[c, pl.ds(g * VL, VL)] = plsc.load_gather(
                tab_v, [seq16, lanes * 0 + c])
    stores = [
        pltpu.async_copy(ohtbuf_v.at[c], oht_hbm.at[c, pl.ds(base, EW)], sem)
        for c in range(N_BASES)
    ]
    for st in stores:
        st.wait()


def _tc_body(ohtb_ref, oht_ref, out_ref):
    ohtb = ohtb_ref[...]
    oht = oht_ref[...]
    for c in range(N_BASES):
        col = jnp.swapaxes(ohtb[c:c + 1, :], 0, 1)
        out_ref[c] = jnp.broadcast_to(col, (BI, L))
        out_ref[N_BASES + c] = jnp.broadcast_to(oht[c:c + 1, :], (BI, L))


def kernel(sequence, base_table):
    sc = functools.partial(
        pl.kernel,
        mesh=plsc.VectorSubcoreMesh(core_axis_name="c", subcore_axis_name="s"),
        compiler_params=pltpu.CompilerParams(needs_layout_passes=False),
        out_type=[jax.ShapeDtypeStruct((N_BASES, L), jnp.float32)],
        scratch_types=[
            pltpu.VMEM((EW,), jnp.int32),
            pltpu.VMEM((N_BASES, N_BASES), jnp.float32),
            pltpu.VMEM((N_BASES, EW), jnp.float32),
            pltpu.SemaphoreType.DMA,
        ],
    )(_sc_gather)
    (oht,) = sc(sequence, base_table)

    out = pl.pallas_call(
        _tc_body,
        grid=(L // BI,),
        in_specs=[
            pl.BlockSpec((N_BASES, BI), lambda i: (0, i)),
            pl.BlockSpec((N_BASES, L), lambda i: (0, 0)),
        ],
        out_specs=pl.BlockSpec((2 * N_BASES, BI, L), lambda i: (0, i, 0)),
        out_shape=jax.ShapeDtypeStruct((2 * N_BASES, L, L), jnp.float32),
    )(oht, oht)
    return out[None]


# speedup vs baseline: 1.0354x; 1.0354x over previous
import functools

import jax
import jax.numpy as jnp
from jax import lax
from jax.experimental import pallas as pl
from jax.experimental.pallas import tpu as pltpu
from jax.experimental.pallas import tpu_sc as plsc

N_BASES = 4
L = 1024
BI = 128
NW = 32
EW = L * N_BASES // NW
VL = 16


def _sc_gather(seq_hbm, tab_hbm, oht_hbm, seq_v, tab_v, ohtbuf_v, sem):
    wid = lax.axis_index("s") * 2 + lax.axis_index("c")
    c = wid % N_BASES
    base = (wid // N_BASES) * EW
    cp_seq = pltpu.async_copy(seq_hbm.at[pl.ds(base, EW)], seq_v, sem)
    cp_tab = pltpu.async_copy(tab_hbm, tab_v, sem)
    cp_seq.wait()
    cp_tab.wait()
    lanes = lax.iota(jnp.int32, VL)
    for g in range(EW // VL):
        seq16 = seq_v[pl.ds(g * VL, VL)]
        ohtbuf_v[pl.ds(g * VL, VL)] = plsc.load_gather(
            tab_v, [seq16, lanes * 0 + c])
    st = pltpu.async_copy(ohtbuf_v, oht_hbm.at[c, pl.ds(base, EW)], sem)
    st.wait()


def _tc_body(ohtb_ref, oht_ref, out_ref):
    ohtb = ohtb_ref[...]
    oht = oht_ref[...]
    for c in range(N_BASES):
        col = jnp.swapaxes(ohtb[c:c + 1, :], 0, 1)
        out_ref[c] = jnp.broadcast_to(col, (BI, L))
        out_ref[N_BASES + c] = jnp.broadcast_to(oht[c:c + 1, :], (BI, L))


def kernel(sequence, base_table):
    sc = functools.partial(
        pl.kernel,
        mesh=plsc.VectorSubcoreMesh(core_axis_name="c", subcore_axis_name="s"),
        compiler_params=pltpu.CompilerParams(needs_layout_passes=False),
        out_type=[jax.ShapeDtypeStruct((N_BASES, L), jnp.float32)],
        scratch_types=[
            pltpu.VMEM((EW,), jnp.int32),
            pltpu.VMEM((N_BASES, N_BASES), jnp.float32),
            pltpu.VMEM((EW,), jnp.float32),
            pltpu.SemaphoreType.DMA,
        ],
    )(_sc_gather)
    (oht,) = sc(sequence, base_table)

    out = pl.pallas_call(
        _tc_body,
        grid=(L // BI,),
        in_specs=[
            pl.BlockSpec((N_BASES, BI), lambda i: (0, i)),
            pl.BlockSpec((N_BASES, L), lambda i: (0, 0)),
        ],
        out_specs=pl.BlockSpec((2 * N_BASES, BI, L), lambda i: (0, i, 0)),
        out_shape=jax.ShapeDtypeStruct((2 * N_BASES, L, L), jnp.float32),
    )(oht, oht)
    return out[None]
